# baseline (speedup 1.0000x reference)
import functools

import jax
import jax.numpy as jnp
from jax import lax
from jax.experimental import pallas as pl
from jax.experimental.pallas import tpu as pltpu
from jax.experimental.pallas import tpu_sc as plsc

N = 10000
E = 320000
DIN = 128
H = 256
DOUT = 2
G = 64

NC = 2
NT = 16
NW = NC * NT
CHUNK = 128
NCH_T = 160
E_PAD = NT * NCH_T * CHUNK
NCH_W = NCH_T // 2
IB = 32
N_PAD = 10240
ROWS_PT = N_PAD // NT
BLK = 512
NBLK = N_PAD // BLK
DEGW = 16

_mesh = functools.partial(plsc.VectorSubcoreMesh,
                          core_axis_name="c", subcore_axis_name="s",
                          num_cores=NC, num_subcores=NT)


def _make_deg_kernel():
    @functools.partial(
        pl.kernel,
        out_type=jax.ShapeDtypeStruct((NC, N_PAD, DEGW), jnp.float32),
        mesh=_mesh(),
        compiler_params=pltpu.CompilerParams(use_tc_tiling_on_sc=False),
        scratch_types=[
            pltpu.VMEM((NCH_W, CHUNK), jnp.int32),
            pltpu.VMEM((CHUNK, DEGW), jnp.float32),
            pltpu.VMEM_SHARED((N_PAD, DEGW), jnp.float32),
        ],
    )
    def deg_kernel(dst_hbm, ones_hbm, out_hbm, idx_dst, ones_v, acc):
        c = lax.axis_index("c")
        s = lax.axis_index("s")
        wid = s * NC + c
        r0 = s * ROWS_PT
        pltpu.sync_copy(dst_hbm.at[wid], idx_dst)
        pltpu.sync_copy(ones_hbm.at[pl.ds(0, CHUNK)], ones_v)
        pltpu.sync_copy(ones_hbm.at[pl.ds(r0, ROWS_PT)],
                        acc.at[pl.ds(r0, ROWS_PT)])
        plsc.subcore_barrier()

        def step(j, carry):
            pltpu.sync_copy(ones_v, acc.at[idx_dst.at[j]], add=True)
            return carry

        lax.fori_loop(0, NCH_W, step, 0)
        plsc.subcore_barrier()
        pltpu.sync_copy(acc.at[pl.ds(r0, ROWS_PT)],
                        out_hbm.at[c, pl.ds(r0, ROWS_PT)])

    return deg_kernel


def _make_mp_kernel():
    @functools.partial(
        pl.kernel,
        out_type=jax.ShapeDtypeStruct((NC, N_PAD, H // 2), jnp.float32),
        mesh=_mesh(),
        scratch_types=[
            pltpu.VMEM((IB, CHUNK), jnp.int32),
            pltpu.VMEM((IB, CHUNK), jnp.int32),
            pltpu.VMEM((CHUNK, H // 2), jnp.float32),
            pltpu.VMEM((CHUNK, H // 2), jnp.float32),
            pltpu.VMEM_SHARED((N_PAD, H // 2), jnp.float32),
            pltpu.SemaphoreType.DMA,
            pltpu.SemaphoreType.DMA,
            pltpu.SemaphoreType.DMA,
            pltpu.SemaphoreType.DMA,
        ],
    )
    def mp_kernel(xs_hbm, src_hbm, dst_hbm, out_hbm,
                  idx_src, idx_dst, buf0, buf1, acc,
                  gsem0, gsem1, ssem0, ssem1):
        c = lax.axis_index("c")
        s = lax.axis_index("s")
        r0 = s * ROWS_PT
        pltpu.sync_copy(xs_hbm.at[pl.ds(c * N_PAD + r0, ROWS_PT)],
                        acc.at[pl.ds(r0, ROWS_PT)])
        plsc.subcore_barrier()

        def blk(jb, carry):
            pltpu.sync_copy(src_hbm.at[c, s, pl.ds(jb * IB, IB)], idx_src)
            pltpu.sync_copy(dst_hbm.at[s, pl.ds(jb * IB, IB)], idx_dst)
            pltpu.async_copy(xs_hbm.at[idx_src.at[0]], buf0, gsem0)
            pltpu.async_copy(xs_hbm.at[idx_src.at[1]], buf1, gsem1)

            def pair(p, carry2):
                j = 2 * p
                pltpu.make_async_copy(xs_hbm.at[idx_src.at[j]],
                                      buf0, gsem0).wait()
                pltpu.async_copy(buf0, acc.at[idx_dst.at[j]], ssem0,
                                 add=True)
                pltpu.make_async_copy(xs_hbm.at[idx_src.at[j + 1]],
                                      buf1, gsem1).wait()
                pltpu.async_copy(buf1, acc.at[idx_dst.at[j + 1]], ssem1,
                                 add=True)

                @pl.when(j + 2 < IB)
                def _refill():
                    pltpu.make_async_copy(buf0, acc.at[idx_dst.at[j]],
                                          ssem0).wait()
                    pltpu.async_copy(xs_hbm.at[idx_src.at[j + 2]],
                                     buf0, gsem0)
                    pltpu.make_async_copy(buf1, acc.at[idx_dst.at[j + 1]],
                                          ssem1).wait()
                    pltpu.async_copy(xs_hbm.at[idx_src.at[j + 3]],
                                     buf1, gsem1)
                return carry2

            lax.fori_loop(0, IB // 2, pair, 0)
            pltpu.make_async_copy(buf0, acc.at[idx_dst.at[IB - 2]],
                                  ssem0).wait()
            pltpu.make_async_copy(buf1, acc.at[idx_dst.at[IB - 1]],
                                  ssem1).wait()
            return carry

        lax.fori_loop(0, NCH_T // IB, blk, 0)
        plsc.subcore_barrier()
        pltpu.sync_copy(acc.at[pl.ds(r0, ROWS_PT)],
                        out_hbm.at[c, pl.ds(r0, ROWS_PT)])

    return mp_kernel


def _p1_call(deg, x_pad, W1):
    def body(deg_ref, x_ref, w_ref, dinv_ref, xsw_ref):
        d = deg_ref[0] + deg_ref[1]
        dv = 1.0 / jnp.sqrt(d[:, 0:1] - 1.0)
        dinv_ref[...] = jnp.broadcast_to(dv, (BLK, DIN))
        hw = jnp.dot(x_ref[...], w_ref[...],
                     preferred_element_type=jnp.float32)
        xsw = hw * dv
        xsw_ref[0] = xsw[:, : H // 2]
        xsw_ref[1] = xsw[:, H // 2:]

    return pl.pallas_call(
        body,
        grid=(NBLK,),
        in_specs=[
            pl.BlockSpec((NC, BLK, DEGW), lambda i: (0, i, 0)),
            pl.BlockSpec((BLK, DIN), lambda i: (i, 0)),
            pl.BlockSpec((DIN, H), lambda i: (0, 0)),
        ],
        out_specs=[
            pl.BlockSpec((BLK, DIN), lambda i: (i, 0)),
            pl.BlockSpec((NC, BLK, H // 2), lambda i: (0, i, 0)),
        ],
        out_shape=[
            jax.ShapeDtypeStruct((N_PAD, DIN), jnp.float32),
            jax.ShapeDtypeStruct((NC, N_PAD, H // 2), jnp.float32),
        ],
    )(deg, x_pad, W1)


def _p2_call(y3, dinv, W2, b1):
    def body(y_ref, dinv_ref, w_ref, b_ref, xsw_ref):
        dv = dinv_ref[...][:, 0:1]
        h1 = jnp.concatenate([y_ref[0], y_ref[1]], axis=1) * dv
        h1 = jnp.maximum(h1 + b_ref[...], 0.0)
        hw2 = jnp.dot(h1, w_ref[...], preferred_element_type=jnp.float32)
        xsw2 = hw2 * dv
        xsw_ref[0] = xsw2[:, : H // 2]
        xsw_ref[1] = xsw2[:, H // 2:]

    return pl.pallas_call(
        body,
        grid=(NBLK,),
        in_specs=[
            pl.BlockSpec((NC, BLK, H // 2), lambda i: (0, i, 0)),
            pl.BlockSpec((BLK, DIN), lambda i: (i, 0)),
            pl.BlockSpec((H, H), lambda i: (0, 0)),
            pl.BlockSpec((1, H), lambda i: (0, 0)),
        ],
        out_specs=pl.BlockSpec((NC, BLK, H // 2), lambda i: (0, i, 0)),
        out_shape=jax.ShapeDtypeStruct((NC, N_PAD, H // 2), jnp.float32),
    )(y3, dinv, W2, b1)


def _p3_call(q3, dinv, b2, batch3, Wo1, bo1, gamma, beta, Wo2, bo2):
    def body(q_ref, dinv_ref, b_ref, batch_ref,
             wo1_ref, bo1_ref, g_ref, be_ref, wo2_ref, bo2_ref,
             out_ref, sums, cnts):
        i = pl.program_id(0)

        @pl.when(i == 0)
        def _init():
            sums[...] = jnp.zeros_like(sums)
            cnts[...] = jnp.zeros_like(cnts)

        dv = dinv_ref[...][:, 0:1]
        h2 = jnp.concatenate([q_ref[0], q_ref[1]], axis=1) * dv
        h2 = jnp.maximum(h2 + b_ref[...], 0.0)
        bb = batch_ref[0, 0]
        onehot = (lax.broadcasted_iota(jnp.int32, (G, BLK), 0)
                  == bb[None, :]).astype(jnp.float32)
        sums[...] += jnp.dot(onehot, h2, preferred_element_type=jnp.float32,
                             precision=lax.Precision.HIGHEST)
        cnts[...] += jnp.broadcast_to(
            jnp.sum(onehot, axis=1, keepdims=True), (G, DIN))

        @pl.when(i == NBLK - 1)
        def _fin():
            pooled = sums[...] / jnp.maximum(cnts[...][:, 0:1], 1.0)
            z = jnp.dot(pooled, wo1_ref[...],
                        preferred_element_type=jnp.float32) + bo1_ref[...]
            mu = jnp.mean(z, axis=0, keepdims=True)
            var = jnp.mean((z - mu) ** 2, axis=0, keepdims=True)
            zn = (z - mu) / jnp.sqrt(var + 1e-5) * g_ref[...] + be_ref[...]
            zn = jnp.maximum(zn, 0.0)
            out_ref[...] = jnp.dot(zn, wo2_ref[...],
                                   preferred_element_type=jnp.float32) \
                + bo2_ref[...]

    return pl.pallas_call(
        body,
        grid=(NBLK,),
        in_specs=[
            pl.BlockSpec((NC, BLK, H // 2), lambda i: (0, i, 0)),
            pl.BlockSpec((BLK, DIN), lambda i: (i, 0)),
            pl.BlockSpec((1, H), lambda i: (0, 0)),
            pl.BlockSpec((1, 1, BLK), lambda i: (i, 0, 0)),
            pl.BlockSpec((H, H), lambda i: (0, 0)),
            pl.BlockSpec((1, H), lambda i: (0, 0)),
            pl.BlockSpec((1, H), lambda i: (0, 0)),
            pl.BlockSpec((1, H), lambda i: (0, 0)),
            pl.BlockSpec((H, DOUT), lambda i: (0, 0)),
            pl.BlockSpec((1, DOUT), lambda i: (0, 0)),
        ],
        out_specs=pl.BlockSpec((G, DOUT), lambda i: (0, 0)),
        out_shape=jax.ShapeDtypeStruct((G, DOUT), jnp.float32),
        scratch_shapes=[
            pltpu.VMEM((G, H), jnp.float32),
            pltpu.VMEM((G, DIN), jnp.float32),
        ],
    )(q3, dinv, b2, batch3, Wo1, bo1, gamma, beta, Wo2, bo2)


def kernel(x, edge_index, batch, W1, b1, W2, b2,
           Wo1, bo1, gamma, beta, Wo2, bo2):
    deg_kernel = _make_deg_kernel()
    mp = _make_mp_kernel()

    pad = E_PAD - E
    fill = jnp.full((pad,), N, jnp.int32)
    srcp = jnp.concatenate([edge_index[0], fill])
    dstp = jnp.concatenate([edge_index[1], fill])
    dstw = dstp.reshape(NW, NCH_W, CHUNK)
    src3 = srcp.reshape(NT, NCH_T, CHUNK)
    src4 = jnp.stack([src3, src3 + N_PAD])
    dst3 = dstp.reshape(NT, NCH_T, CHUNK)
    ones_h = jnp.ones((N_PAD, DEGW), jnp.float32)
    x_pad = jnp.pad(x, ((0, N_PAD - N), (0, 0)))
    batch3 = jnp.pad(batch, (0, N_PAD - N),
                     constant_values=G).reshape(NBLK, 1, BLK)

    deg = deg_kernel(dstw, ones_h)
    dinv, xsw = _p1_call(deg, x_pad, W1)
    y3 = mp(xsw.reshape(NC * N_PAD, H // 2), src4, dst3)
    xsw2 = _p2_call(y3, dinv, W2, b1.reshape(1, H))
    q3 = mp(xsw2.reshape(NC * N_PAD, H // 2), src4, dst3)
    out = _p3_call(q3, dinv, b2.reshape(1, H), batch3,
                   Wo1, bo1.reshape(1, H), gamma.reshape(1, H),
                   beta.reshape(1, H), Wo2, bo2.reshape(1, DOUT))
    return out


# speedup vs baseline: 1.0854x; 1.0854x over previous
import functools

import jax
import jax.numpy as jnp
from jax import lax
from jax.experimental import pallas as pl
from jax.experimental.pallas import tpu as pltpu
from jax.experimental.pallas import tpu_sc as plsc

N = 10000
E = 320000
DIN = 128
H = 256
DOUT = 2
G = 64

NC = 2
NT = 16
NW = NC * NT
CHUNK = 128
NCH_T = 160
E_PAD = NT * NCH_T * CHUNK
NCH_W = NCH_T // 2
IB = 32
N_PAD = 10240
ROWS_PT = N_PAD // NT
BLK = 512
NBLK = N_PAD // BLK
DEGW = 16

_mesh = functools.partial(plsc.VectorSubcoreMesh,
                          core_axis_name="c", subcore_axis_name="s",
                          num_cores=NC, num_subcores=NT)


def _make_deg_kernel():
    @functools.partial(
        pl.kernel,
        out_type=jax.ShapeDtypeStruct((NC, N_PAD, DEGW), jnp.float32),
        mesh=_mesh(),
        compiler_params=pltpu.CompilerParams(use_tc_tiling_on_sc=False),
        scratch_types=[
            pltpu.VMEM((NCH_W, CHUNK), jnp.int32),
            pltpu.VMEM((CHUNK, DEGW), jnp.float32),
            pltpu.VMEM_SHARED((N_PAD, DEGW), jnp.float32),
        ],
    )
    def deg_kernel(dst_hbm, ones_hbm, out_hbm, idx_dst, ones_v, acc):
        c = lax.axis_index("c")
        s = lax.axis_index("s")
        wid = s * NC + c
        r0 = s * ROWS_PT
        pltpu.sync_copy(dst_hbm.at[wid], idx_dst)
        pltpu.sync_copy(ones_hbm.at[pl.ds(0, CHUNK)], ones_v)
        pltpu.sync_copy(ones_hbm.at[pl.ds(r0, ROWS_PT)],
                        acc.at[pl.ds(r0, ROWS_PT)])
        plsc.subcore_barrier()

        def step(j, carry):
            pltpu.sync_copy(ones_v, acc.at[idx_dst.at[j]], add=True)
            return carry

        lax.fori_loop(0, NCH_W, step, 0)
        plsc.subcore_barrier()
        pltpu.sync_copy(acc.at[pl.ds(r0, ROWS_PT)],
                        out_hbm.at[c, pl.ds(r0, ROWS_PT)])

    return deg_kernel


def _make_mp_kernel():
    @functools.partial(
        pl.kernel,
        out_type=jax.ShapeDtypeStruct((NC, N_PAD, H // 2), jnp.float32),
        mesh=_mesh(),
        scratch_types=[
            pltpu.VMEM((IB, CHUNK), jnp.int32),
            pltpu.VMEM((IB, CHUNK), jnp.int32),
            pltpu.VMEM((CHUNK, H // 2), jnp.float32),
            pltpu.VMEM((CHUNK, H // 2), jnp.float32),
            pltpu.VMEM_SHARED((N_PAD, H // 2), jnp.float32),
            pltpu.SemaphoreType.DMA,
            pltpu.SemaphoreType.DMA,
        ],
    )
    def mp_kernel(xs_hbm, src_hbm, dst_hbm, out_hbm,
                  idx_src, idx_dst, buf0, buf1, acc, gsem0, gsem1):
        c = lax.axis_index("c")
        s = lax.axis_index("s")
        r0 = s * ROWS_PT
        pltpu.sync_copy(xs_hbm.at[pl.ds(c * N_PAD + r0, ROWS_PT)],
                        acc.at[pl.ds(r0, ROWS_PT)])
        plsc.subcore_barrier()

        def blk(jb, carry):
            pltpu.sync_copy(src_hbm.at[c, s, pl.ds(jb * IB, IB)], idx_src)
            pltpu.sync_copy(dst_hbm.at[s, pl.ds(jb * IB, IB)], idx_dst)
            pltpu.async_copy(xs_hbm.at[idx_src.at[0]], buf0, gsem0)
            pltpu.async_copy(xs_hbm.at[idx_src.at[1]], buf1, gsem1)

            def pair(p, carry2):
                j = 2 * p
                pltpu.make_async_copy(xs_hbm.at[idx_src.at[j]],
                                      buf0, gsem0).wait()
                pltpu.sync_copy(buf0, acc.at[idx_dst.at[j]], add=True)

                @pl.when(j + 2 < IB)
                def _prefetch0():
                    pltpu.async_copy(xs_hbm.at[idx_src.at[j + 2]],
                                     buf0, gsem0)

                pltpu.make_async_copy(xs_hbm.at[idx_src.at[j + 1]],
                                      buf1, gsem1).wait()
                pltpu.sync_copy(buf1, acc.at[idx_dst.at[j + 1]], add=True)

                @pl.when(j + 3 < IB)
                def _prefetch1():
                    pltpu.async_copy(xs_hbm.at[idx_src.at[j + 3]],
                                     buf1, gsem1)
                return carry2

            lax.fori_loop(0, IB // 2, pair, 0)
            return carry

        lax.fori_loop(0, NCH_T // IB, blk, 0)
        plsc.subcore_barrier()
        pltpu.sync_copy(acc.at[pl.ds(r0, ROWS_PT)],
                        out_hbm.at[c, pl.ds(r0, ROWS_PT)])

    return mp_kernel


def _p1_call(deg, x_pad, W1):
    def body(deg_ref, x_ref, w_ref, dinv_ref, xsw_ref):
        d = deg_ref[0] + deg_ref[1]
        dv = 1.0 / jnp.sqrt(d[:, 0:1] - 1.0)
        dinv_ref[...] = jnp.broadcast_to(dv, (BLK, DIN))
        hw = jnp.dot(x_ref[...], w_ref[...],
                     preferred_element_type=jnp.float32)
        xsw = hw * dv
        xsw_ref[0] = xsw[:, : H // 2]
        xsw_ref[1] = xsw[:, H // 2:]

    return pl.pallas_call(
        body,
        grid=(NBLK,),
        in_specs=[
            pl.BlockSpec((NC, BLK, DEGW), lambda i: (0, i, 0)),
            pl.BlockSpec((BLK, DIN), lambda i: (i, 0)),
            pl.BlockSpec((DIN, H), lambda i: (0, 0)),
        ],
        out_specs=[
            pl.BlockSpec((BLK, DIN), lambda i: (i, 0)),
            pl.BlockSpec((NC, BLK, H // 2), lambda i: (0, i, 0)),
        ],
        out_shape=[
            jax.ShapeDtypeStruct((N_PAD, DIN), jnp.float32),
            jax.ShapeDtypeStruct((NC, N_PAD, H // 2), jnp.float32),
        ],
    )(deg, x_pad, W1)


def _p2_call(y3, dinv, W2, b1):
    def body(y_ref, dinv_ref, w_ref, b_ref, xsw_ref):
        dv = dinv_ref[...][:, 0:1]
        h1 = jnp.concatenate([y_ref[0], y_ref[1]], axis=1) * dv
        h1 = jnp.maximum(h1 + b_ref[...], 0.0)
        hw2 = jnp.dot(h1, w_ref[...], preferred_element_type=jnp.float32)
        xsw2 = hw2 * dv
        xsw_ref[0] = xsw2[:, : H // 2]
        xsw_ref[1] = xsw2[:, H // 2:]

    return pl.pallas_call(
        body,
        grid=(NBLK,),
        in_specs=[
            pl.BlockSpec((NC, BLK, H // 2), lambda i: (0, i, 0)),
            pl.BlockSpec((BLK, DIN), lambda i: (i, 0)),
            pl.BlockSpec((H, H), lambda i: (0, 0)),
            pl.BlockSpec((1, H), lambda i: (0, 0)),
        ],
        out_specs=pl.BlockSpec((NC, BLK, H // 2), lambda i: (0, i, 0)),
        out_shape=jax.ShapeDtypeStruct((NC, N_PAD, H // 2), jnp.float32),
    )(y3, dinv, W2, b1)


def _p3_call(q3, dinv, b2, batch3, Wo1, bo1, gamma, beta, Wo2, bo2):
    def body(q_ref, dinv_ref, b_ref, batch_ref,
             wo1_ref, bo1_ref, g_ref, be_ref, wo2_ref, bo2_ref,
             out_ref, sums, cnts):
        i = pl.program_id(0)

        @pl.when(i == 0)
        def _init():
            sums[...] = jnp.zeros_like(sums)
            cnts[...] = jnp.zeros_like(cnts)

        dv = dinv_ref[...][:, 0:1]
        h2 = jnp.concatenate([q_ref[0], q_ref[1]], axis=1) * dv
        h2 = jnp.maximum(h2 + b_ref[...], 0.0)
        bb = batch_ref[0, 0]
        onehot = (lax.broadcasted_iota(jnp.int32, (G, BLK), 0)
                  == bb[None, :]).astype(jnp.float32)
        sums[...] += jnp.dot(onehot, h2, preferred_element_type=jnp.float32,
                             precision=lax.Precision.HIGHEST)
        cnts[...] += jnp.broadcast_to(
            jnp.sum(onehot, axis=1, keepdims=True), (G, DIN))

        @pl.when(i == NBLK - 1)
        def _fin():
            pooled = sums[...] / jnp.maximum(cnts[...][:, 0:1], 1.0)
            z = jnp.dot(pooled, wo1_ref[...],
                        preferred_element_type=jnp.float32) + bo1_ref[...]
            mu = jnp.mean(z, axis=0, keepdims=True)
            var = jnp.mean((z - mu) ** 2, axis=0, keepdims=True)
            zn = (z - mu) / jnp.sqrt(var + 1e-5) * g_ref[...] + be_ref[...]
            zn = jnp.maximum(zn, 0.0)
            out_ref[...] = jnp.dot(zn, wo2_ref[...],
                                   preferred_element_type=jnp.float32) \
                + bo2_ref[...]

    return pl.pallas_call(
        body,
        grid=(NBLK,),
        in_specs=[
            pl.BlockSpec((NC, BLK, H // 2), lambda i: (0, i, 0)),
            pl.BlockSpec((BLK, DIN), lambda i: (i, 0)),
            pl.BlockSpec((1, H), lambda i: (0, 0)),
            pl.BlockSpec((1, 1, BLK), lambda i: (i, 0, 0)),
            pl.BlockSpec((H, H), lambda i: (0, 0)),
            pl.BlockSpec((1, H), lambda i: (0, 0)),
            pl.BlockSpec((1, H), lambda i: (0, 0)),
            pl.BlockSpec((1, H), lambda i: (0, 0)),
            pl.BlockSpec((H, DOUT), lambda i: (0, 0)),
            pl.BlockSpec((1, DOUT), lambda i: (0, 0)),
        ],
        out_specs=pl.BlockSpec((G, DOUT), lambda i: (0, 0)),
        out_shape=jax.ShapeDtypeStruct((G, DOUT), jnp.float32),
        scratch_shapes=[
            pltpu.VMEM((G, H), jnp.float32),
            pltpu.VMEM((G, DIN), jnp.float32),
        ],
    )(q3, dinv, b2, batch3, Wo1, bo1, gamma, beta, Wo2, bo2)


def kernel(x, edge_index, batch, W1, b1, W2, b2,
           Wo1, bo1, gamma, beta, Wo2, bo2):
    deg_kernel = _make_deg_kernel()
    mp = _make_mp_kernel()

    pad = E_PAD - E
    fill = jnp.full((pad,), N, jnp.int32)
    srcp = jnp.concatenate([edge_index[0], fill])
    dstp = jnp.concatenate([edge_index[1], fill])
    dstw = dstp.reshape(NW, NCH_W, CHUNK)
    src3 = srcp.reshape(NT, NCH_T, CHUNK)
    src4 = jnp.stack([src3, src3 + N_PAD])
    dst3 = dstp.reshape(NT, NCH_T, CHUNK)
    ones_h = jnp.ones((N_PAD, DEGW), jnp.float32)
    x_pad = jnp.pad(x, ((0, N_PAD - N), (0, 0)))
    batch3 = jnp.pad(batch, (0, N_PAD - N),
                     constant_values=G).reshape(NBLK, 1, BLK)

    deg = deg_kernel(dstw, ones_h)
    dinv, xsw = _p1_call(deg, x_pad, W1)
    y3 = mp(xsw.reshape(NC * N_PAD, H // 2), src4, dst3)
    xsw2 = _p2_call(y3, dinv, W2, b1.reshape(1, H))
    q3 = mp(xsw2.reshape(NC * N_PAD, H // 2), src4, dst3)
    out = _p3_call(q3, dinv, b2.reshape(1, H), batch3,
                   Wo1, bo1.reshape(1, H), gamma.reshape(1, H),
                   beta.reshape(1, H), Wo2, bo2.reshape(1, DOUT))
    return out


# speedup vs baseline: 1.0873x; 1.0018x over previous
import functools

import jax
import jax.numpy as jnp
from jax import lax
from jax.experimental import pallas as pl
from jax.experimental.pallas import tpu as pltpu
from jax.experimental.pallas import tpu_sc as plsc

N = 10000
E = 320000
DIN = 128
H = 256
DOUT = 2
G = 64

NC = 2
NT = 16
NW = NC * NT
CHUNK = 128
NCH_T = 160
E_PAD = NT * NCH_T * CHUNK
IB = 32
CHUNK_D = 128
NCH_W = 80
E_PAD_D = NW * NCH_W * CHUNK_D
N_PAD = 10240
ROWS_PT = N_PAD // NT
BLK = 512
NBLK = N_PAD // BLK
DEGW = 16

_mesh = functools.partial(plsc.VectorSubcoreMesh,
                          core_axis_name="c", subcore_axis_name="s",
                          num_cores=NC, num_subcores=NT)


def _make_deg_kernel():
    @functools.partial(
        pl.kernel,
        out_type=jax.ShapeDtypeStruct((NC, N_PAD, DEGW), jnp.float32),
        mesh=_mesh(),
        compiler_params=pltpu.CompilerParams(use_tc_tiling_on_sc=False),
        scratch_types=[
            pltpu.VMEM((NCH_W, CHUNK_D), jnp.int32),
            pltpu.VMEM((CHUNK_D, DEGW), jnp.float32),
            pltpu.VMEM_SHARED((N_PAD, DEGW), jnp.float32),
        ],
    )
    def deg_kernel(dst_hbm, ones_hbm, out_hbm, idx_dst, ones_v, acc):
        c = lax.axis_index("c")
        s = lax.axis_index("s")
        wid = s * NC + c
        r0 = s * ROWS_PT
        pltpu.sync_copy(dst_hbm.at[wid], idx_dst)
        pltpu.sync_copy(ones_hbm.at[pl.ds(0, CHUNK_D)], ones_v)
        pltpu.sync_copy(ones_hbm.at[pl.ds(r0, ROWS_PT)],
                        acc.at[pl.ds(r0, ROWS_PT)])
        plsc.subcore_barrier()

        def step(j, carry):
            pltpu.sync_copy(ones_v, acc.at[idx_dst.at[j]], add=True)
            return carry

        lax.fori_loop(0, NCH_W, step, 0)
        plsc.subcore_barrier()
        pltpu.sync_copy(acc.at[pl.ds(r0, ROWS_PT)],
                        out_hbm.at[c, pl.ds(r0, ROWS_PT)])

    return deg_kernel


def _make_mp_kernel():
    @functools.partial(
        pl.kernel,
        out_type=jax.ShapeDtypeStruct((NC, N_PAD, H // 2), jnp.float32),
        mesh=_mesh(),
        scratch_types=[
            pltpu.VMEM((IB, CHUNK), jnp.int32),
            pltpu.VMEM((IB, CHUNK), jnp.int32),
            pltpu.VMEM((CHUNK, H // 2), jnp.float32),
            pltpu.VMEM((CHUNK, H // 2), jnp.float32),
            pltpu.VMEM_SHARED((N_PAD, H // 2), jnp.float32),
            pltpu.SemaphoreType.DMA,
            pltpu.SemaphoreType.DMA,
        ],
    )
    def mp_kernel(xs_hbm, src_hbm, dst_hbm, out_hbm,
                  idx_src, idx_dst, buf0, buf1, acc, gsem0, gsem1):
        c = lax.axis_index("c")
        s = lax.axis_index("s")
        r0 = s * ROWS_PT
        pltpu.sync_copy(xs_hbm.at[pl.ds(c * N_PAD + r0, ROWS_PT)],
                        acc.at[pl.ds(r0, ROWS_PT)])
        plsc.subcore_barrier()

        def blk(jb, carry):
            pltpu.sync_copy(src_hbm.at[c, s, pl.ds(jb * IB, IB)], idx_src)
            pltpu.sync_copy(dst_hbm.at[s, pl.ds(jb * IB, IB)], idx_dst)
            pltpu.async_copy(xs_hbm.at[idx_src.at[0]], buf0, gsem0)
            pltpu.async_copy(xs_hbm.at[idx_src.at[1]], buf1, gsem1)

            def pair(p, carry2):
                j = 2 * p
                pltpu.make_async_copy(xs_hbm.at[idx_src.at[j]],
                                      buf0, gsem0).wait()
                pltpu.sync_copy(buf0, acc.at[idx_dst.at[j]], add=True)

                @pl.when(j + 2 < IB)
                def _prefetch0():
                    pltpu.async_copy(xs_hbm.at[idx_src.at[j + 2]],
                                     buf0, gsem0)

                pltpu.make_async_copy(xs_hbm.at[idx_src.at[j + 1]],
                                      buf1, gsem1).wait()
                pltpu.sync_copy(buf1, acc.at[idx_dst.at[j + 1]], add=True)

                @pl.when(j + 3 < IB)
                def _prefetch1():
                    pltpu.async_copy(xs_hbm.at[idx_src.at[j + 3]],
                                     buf1, gsem1)
                return carry2

            lax.fori_loop(0, IB // 2, pair, 0)
            return carry

        lax.fori_loop(0, NCH_T // IB, blk, 0)
        plsc.subcore_barrier()
        pltpu.sync_copy(acc.at[pl.ds(r0, ROWS_PT)],
                        out_hbm.at[c, pl.ds(r0, ROWS_PT)])

    return mp_kernel


def _p1_call(deg, x_pad, W1):
    def body(deg_ref, x_ref, w_ref, dinv_ref, xsw_ref):
        d = deg_ref[0] + deg_ref[1]
        dv = 1.0 / jnp.sqrt(d[:, 0:1] - 1.0)
        dinv_ref[...] = jnp.broadcast_to(dv, (BLK, DIN))
        hw = jnp.dot(x_ref[...], w_ref[...],
                     preferred_element_type=jnp.float32)
        xsw = hw * dv
        xsw_ref[0] = xsw[:, : H // 2]
        xsw_ref[1] = xsw[:, H // 2:]

    return pl.pallas_call(
        body,
        grid=(NBLK,),
        in_specs=[
            pl.BlockSpec((NC, BLK, DEGW), lambda i: (0, i, 0)),
            pl.BlockSpec((BLK, DIN), lambda i: (i, 0)),
            pl.BlockSpec((DIN, H), lambda i: (0, 0)),
        ],
        out_specs=[
            pl.BlockSpec((BLK, DIN), lambda i: (i, 0)),
            pl.BlockSpec((NC, BLK, H // 2), lambda i: (0, i, 0)),
        ],
        out_shape=[
            jax.ShapeDtypeStruct((N_PAD, DIN), jnp.float32),
            jax.ShapeDtypeStruct((NC, N_PAD, H // 2), jnp.float32),
        ],
    )(deg, x_pad, W1)


def _p2_call(y3, dinv, W2, b1):
    def body(y_ref, dinv_ref, w_ref, b_ref, xsw_ref):
        dv = dinv_ref[...][:, 0:1]
        h1 = jnp.concatenate([y_ref[0], y_ref[1]], axis=1) * dv
        h1 = jnp.maximum(h1 + b_ref[...], 0.0)
        hw2 = jnp.dot(h1, w_ref[...], preferred_element_type=jnp.float32)
        xsw2 = hw2 * dv
        xsw_ref[0] = xsw2[:, : H // 2]
        xsw_ref[1] = xsw2[:, H // 2:]

    return pl.pallas_call(
        body,
        grid=(NBLK,),
        in_specs=[
            pl.BlockSpec((NC, BLK, H // 2), lambda i: (0, i, 0)),
            pl.BlockSpec((BLK, DIN), lambda i: (i, 0)),
            pl.BlockSpec((H, H), lambda i: (0, 0)),
            pl.BlockSpec((1, H), lambda i: (0, 0)),
        ],
        out_specs=pl.BlockSpec((NC, BLK, H // 2), lambda i: (0, i, 0)),
        out_shape=jax.ShapeDtypeStruct((NC, N_PAD, H // 2), jnp.float32),
    )(y3, dinv, W2, b1)


def _p3_call(q3, dinv, b2, batch3, Wo1, bo1, gamma, beta, Wo2, bo2):
    def body(q_ref, dinv_ref, b_ref, batch_ref,
             wo1_ref, bo1_ref, g_ref, be_ref, wo2_ref, bo2_ref,
             out_ref, sums, cnts):
        i = pl.program_id(0)

        @pl.when(i == 0)
        def _init():
            sums[...] = jnp.zeros_like(sums)
            cnts[...] = jnp.zeros_like(cnts)

        dv = dinv_ref[...][:, 0:1]
        h2 = jnp.concatenate([q_ref[0], q_ref[1]], axis=1) * dv
        h2 = jnp.maximum(h2 + b_ref[...], 0.0)
        bb = batch_ref[0, 0]
        onehot = (lax.broadcasted_iota(jnp.int32, (G, BLK), 0)
                  == bb[None, :]).astype(jnp.float32)
        sums[...] += jnp.dot(onehot, h2, preferred_element_type=jnp.float32,
                             precision=lax.Precision.HIGHEST)
        cnts[...] += jnp.broadcast_to(
            jnp.sum(onehot, axis=1, keepdims=True), (G, DIN))

        @pl.when(i == NBLK - 1)
        def _fin():
            pooled = sums[...] / jnp.maximum(cnts[...][:, 0:1], 1.0)
            z = jnp.dot(pooled, wo1_ref[...],
                        preferred_element_type=jnp.float32) + bo1_ref[...]
            mu = jnp.mean(z, axis=0, keepdims=True)
            var = jnp.mean((z - mu) ** 2, axis=0, keepdims=True)
            zn = (z - mu) / jnp.sqrt(var + 1e-5) * g_ref[...] + be_ref[...]
            zn = jnp.maximum(zn, 0.0)
            out_ref[...] = jnp.dot(zn, wo2_ref[...],
                                   preferred_element_type=jnp.float32) \
                + bo2_ref[...]

    return pl.pallas_call(
        body,
        grid=(NBLK,),
        in_specs=[
            pl.BlockSpec((NC, BLK, H // 2), lambda i: (0, i, 0)),
            pl.BlockSpec((BLK, DIN), lambda i: (i, 0)),
            pl.BlockSpec((1, H), lambda i: (0, 0)),
            pl.BlockSpec((1, 1, BLK), lambda i: (i, 0, 0)),
            pl.BlockSpec((H, H), lambda i: (0, 0)),
            pl.BlockSpec((1, H), lambda i: (0, 0)),
            pl.BlockSpec((1, H), lambda i: (0, 0)),
            pl.BlockSpec((1, H), lambda i: (0, 0)),
            pl.BlockSpec((H, DOUT), lambda i: (0, 0)),
            pl.BlockSpec((1, DOUT), lambda i: (0, 0)),
        ],
        out_specs=pl.BlockSpec((G, DOUT), lambda i: (0, 0)),
        out_shape=jax.ShapeDtypeStruct((G, DOUT), jnp.float32),
        scratch_shapes=[
            pltpu.VMEM((G, H), jnp.float32),
            pltpu.VMEM((G, DIN), jnp.float32),
        ],
    )(q3, dinv, b2, batch3, Wo1, bo1, gamma, beta, Wo2, bo2)


def kernel(x, edge_index, batch, W1, b1, W2, b2,
           Wo1, bo1, gamma, beta, Wo2, bo2):
    deg_kernel = _make_deg_kernel()
    mp = _make_mp_kernel()

    fill = jnp.full((E_PAD - E,), N, jnp.int32)
    srcp = jnp.concatenate([edge_index[0], fill])
    dstp = jnp.concatenate([edge_index[1], fill])
    src3 = srcp.reshape(NT, NCH_T, CHUNK)
    src4 = jnp.stack([src3, src3 + N_PAD])
    dst3 = dstp.reshape(NT, NCH_T, CHUNK)
    fill_d = jnp.full((E_PAD_D - E,), N, jnp.int32)
    dstw = jnp.concatenate([edge_index[1], fill_d]).reshape(
        NW, NCH_W, CHUNK_D)
    ones_h = jnp.ones((N_PAD, DEGW), jnp.float32)
    x_pad = jnp.pad(x, ((0, N_PAD - N), (0, 0)))
    batch3 = jnp.pad(batch, (0, N_PAD - N),
                     constant_values=G).reshape(NBLK, 1, BLK)

    deg = deg_kernel(dstw, ones_h)
    dinv, xsw = _p1_call(deg, x_pad, W1)
    y3 = mp(xsw.reshape(NC * N_PAD, H // 2), src4, dst3)
    xsw2 = _p2_call(y3, dinv, W2, b1.reshape(1, H))
    q3 = mp(xsw2.reshape(NC * N_PAD, H // 2), src4, dst3)
    out = _p3_call(q3, dinv, b2.reshape(1, H), batch3,
                   Wo1, bo1.reshape(1, H), gamma.reshape(1, H),
                   beta.reshape(1, H), Wo2, bo2.reshape(1, DOUT))
    return out
